# baseline (speedup 1.0000x reference)
import functools
import jax
import jax.numpy as jnp
from jax import lax
from jax.experimental import pallas as pl
from jax.experimental.pallas import tpu as pltpu
from jax.experimental.pallas import tpu_sc as plsc

_R = 224
_IDX = 112


def _make_sc_kernel(N, S, HWD, E, dtype):
    info = plsc.get_sparse_core_info()
    NC, NS = info.num_cores, info.num_subcores
    NW = NC * NS
    assert S % NW == 0
    s_per_w = S // NW
    n_chunks = HWD // _R
    mesh = plsc.VectorSubcoreMesh(core_axis_name="c", subcore_axis_name="s")

    @functools.partial(
        pl.kernel,
        mesh=mesh,
        out_type=jax.ShapeDtypeStruct((N, S, HWD, E), dtype),
        scratch_types=[
            pltpu.VMEM((_R, E), dtype),
            pltpu.VMEM((_R, E), dtype),
            pltpu.VMEM((_IDX,), jnp.int32),
            pltpu.VMEM((_IDX,), jnp.int32),
            pltpu.SemaphoreType.DMA((2,)),
            pltpu.SemaphoreType.DMA((2,)),
        ],
    )
    def sc_kernel(table_hbm, out_hbm, buf0, buf1, idx0, idx1, gsem, ssem):
        wid = lax.axis_index("s") * NC + lax.axis_index("c")
        bufs = (buf0, buf1)
        idxs = (idx0, idx1)
        gathers = [[], []]
        scatters = [[], []]
        svals = []
        for k in range(s_per_w):
            s = wid * s_per_w + k
            svals.append(s)
            splat = jnp.full((16,), s, jnp.int32)
            for i in range(_IDX // 16):
                idxs[k][pl.ds(i * 16, 16)] = splat
            for h in range(_R // _IDX):
                cp = pltpu.make_async_copy(
                    table_hbm.at[idxs[k]],
                    bufs[k].at[pl.ds(h * _IDX, _IDX)],
                    gsem.at[k],
                )
                cp.start()
                gathers[k].append(cp)
        for k in range(s_per_w):
            for cp in gathers[k]:
                cp.wait()
            for n in range(N):
                for j in range(n_chunks):
                    cp = pltpu.make_async_copy(
                        bufs[k],
                        out_hbm.at[n, svals[k], pl.ds(j * _R, _R)],
                        ssem.at[k],
                    )
                    cp.start()
                    scatters[k].append(cp)
        for k in range(s_per_w):
            for cp in scatters[k]:
                cp.wait()

    return sc_kernel


def kernel(x, table):
    N, S, H, W, D = x.shape
    T, E = table.shape
    HWD = H * W * D
    out = _make_sc_kernel(N, S, HWD, E, table.dtype)(table)
    return out.reshape(N, S, H, W, D, E)
